# baseline (speedup 1.0000x reference)
import functools
import math

import jax
import jax.numpy as jnp
from jax import lax
from jax.experimental import pallas as pl
from jax.experimental.pallas import tpu as pltpu
from jax.experimental.pallas import tpu_sc as plsc

VOCAB = 1000000
EMBED_DIM = 64
SCALE = math.sqrt(EMBED_DIM)

_INFO = plsc.get_sparse_core_info()
NC = _INFO.num_cores
NS = _INFO.num_subcores
NW = NC * NS
LANES = _INFO.num_lanes

SEQ = 4096
TOK = 200
B_TOTAL = SEQ * TOK
B_PER_W = B_TOTAL // NW

PAIR_ROWS = VOCAB // 2

N_COLS = VOCAB // 128
TAIL_V0 = N_COLS * 128
COLS_PER_W = (N_COLS + NW - 1) // NW

CHUNK = 128
N_CHUNKS = B_PER_W // CHUNK
NBUF = 2
NTR = 2
LOOKAHEAD = 2

S_TILES = SEQ // 128
D_TILES = EMBED_DIM // 8


def _make_detile_kernel():
    mesh = plsc.VectorSubcoreMesh(core_axis_name="c", subcore_axis_name="s")

    @functools.partial(
        pl.kernel,
        out_type=jax.ShapeDtypeStruct((PAIR_ROWS, 128), jnp.float32),
        mesh=mesh,
        scratch_types=[
            [pltpu.VMEM((EMBED_DIM, 128), jnp.float32) for _ in range(2)],
            [pltpu.VMEM((EMBED_DIM, 128), jnp.float32) for _ in range(2)],
            [pltpu.SemaphoreType.DMA for _ in range(2)],
            [pltpu.SemaphoreType.DMA for _ in range(2)],
            pltpu.SemaphoreType.DMA,
        ],
        compiler_params=pltpu.CompilerParams(use_tc_tiling_on_sc=True,
                                             needs_layout_passes=False),
    )
    def detile_kernel(tabt_hbm, tail_hbm, tpad_hbm, buf, tbuf, sem_r, sem_w,
                      sem_t):
        wid = lax.axis_index("s") * NC + lax.axis_index("c")
        lane_iota = lax.iota(jnp.int32, 16)
        row_ids = [d0 * 16 + lane_iota for d0 in range(4)]

        def col(k):
            return wid + k * NW

        def rd(k, b):
            return pltpu.make_async_copy(
                tabt_hbm.at[:, pl.ds(col(k) * 128, 128)], buf[b], sem_r[b])

        def wr(k, b):
            return pltpu.make_async_copy(
                tbuf[b], tpad_hbm.at[pl.ds(col(k) * 64, 64)], sem_w[b])

        def start_read(k, b):
            @pl.when(col(k) < N_COLS)
            def _():
                rd(k, b).start()

        start_read(0, 0)

        @pl.loop(0, COLS_PER_W, step=2)
        def step(i):
            for db in range(2):
                k = i + db
                b = db

                @pl.when(col(k) < N_COLS)
                def _():
                    start_read(k + 1, 1 - b)
                    rd(k, b).wait()

                    @pl.when(k >= 2)
                    def _():
                        wr(k - 2, b).wait()

                    @plsc.parallel_loop(0, 64, unroll=2)
                    def tp(p):
                        col_even = jnp.zeros((16,), jnp.int32) + 2 * p
                        for kk in range(8):
                            h = kk // 4
                            d0 = (kk % 4) * 16
                            vec = plsc.load_gather(
                                buf[b], [row_ids[d0 // 16], col_even + h])
                            tbuf[b][p, pl.ds(kk * 16, 16)] = vec

                    wr(k, b).start()

        for j in range(2):
            k = COLS_PER_W - 2 + j

            @pl.when(col(k) < N_COLS)
            def _():
                wr(k, k % 2).wait()

        @pl.when(wid == 0)
        def _():
            pltpu.async_copy(tail_hbm, tbuf[0].at[pl.ds(0, 32)], sem_t).wait()
            pltpu.async_copy(tbuf[0].at[pl.ds(0, 32)],
                             tpad_hbm.at[pl.ds(TAIL_V0 // 2, 32)],
                             sem_t).wait()

    return detile_kernel


def _make_gather_kernel():
    mesh = plsc.VectorSubcoreMesh(core_axis_name="c", subcore_axis_name="s")

    @functools.partial(
        pl.kernel,
        out_type=jax.ShapeDtypeStruct((TOK, D_TILES, S_TILES, 8, 128),
                                      jnp.float32),
        mesh=mesh,
        scratch_types=[
            pltpu.VMEM((B_PER_W,), jnp.int32),
            [pltpu.VMEM((CHUNK, 128), jnp.float32) for _ in range(NBUF)],
            [pltpu.VMEM((CHUNK,), jnp.int32) for _ in range(NBUF)],
            [pltpu.VMEM((D_TILES, 8, 128), jnp.float32) for _ in range(NTR)],
            [pltpu.SemaphoreType.DMA for _ in range(NBUF)],
            [pltpu.SemaphoreType.DMA for _ in range(NTR)],
            pltpu.SemaphoreType.DMA,
        ],
        compiler_params=pltpu.CompilerParams(use_tc_tiling_on_sc=True,
                                             needs_layout_passes=False),
    )
    def gather_kernel(xt_hbm, tpad_hbm, out_hbm, idx_v, rows, idxh, tr,
                      sem_g, sem_s, sem_i):
        wid = lax.axis_index("s") * NC + lax.axis_index("c")
        base = wid * B_PER_W

        pltpu.async_copy(xt_hbm.at[pl.ds(base, B_PER_W)], idx_v, sem_i).wait()

        lane_iota = lax.iota(jnp.int32, 16)
        row_ids = [sl0 * 16 + lane_iota for sl0 in range(8)]

        def out_slice(c):
            j0 = base + c * CHUNK
            t = j0 // SEQ
            sh = (j0 - t * SEQ) // 128
            return out_hbm.at[t, :, sh]

        def start_gather(c, b):
            @plsc.parallel_loop(0, CHUNK // 16, unroll=2)
            def halve(g):
                v = idx_v[pl.ds(c * CHUNK + g * 16, 16)]
                idxh[b][pl.ds(g * 16, 16)] = lax.shift_right_logical(v, 1)

            pltpu.async_copy(tpad_hbm.at[idxh[b]], rows[b], sem_g[b])

        def wait_gather(c, b):
            pltpu.make_async_copy(tpad_hbm.at[idxh[b]], rows[b],
                                  sem_g[b]).wait()

        def start_scatter(c, q):
            pltpu.async_copy(tr[q], out_slice(c), sem_s[q])

        def wait_scatter(c, q):
            pltpu.make_async_copy(tr[q], out_slice(c), sem_s[q]).wait()

        for c in range(NBUF):
            start_gather(c, c % NBUF)

        @pl.loop(0, N_CHUNKS, step=NBUF)
        def chunk_group(i):
            for db in range(NBUF):
                c = i + db
                b = db
                q = db % NTR

                wait_gather(c, b)

                @pl.when(c >= NTR)
                def _():
                    wait_scatter(c - NTR, q)

                for sl0 in range(8):
                    rids = row_ids[sl0]
                    v = idx_v[pl.ds(c * CHUNK + sl0 * 16, 16)]
                    par_col = lax.shift_left(lax.bitwise_and(v, 1), 6)
                    for d in range(EMBED_DIM):
                        vec = plsc.load_gather(rows[b], [rids, par_col + d])
                        tr[q][d // 8, d % 8,
                              pl.ds(sl0 * 16, 16)] = vec * SCALE

                start_scatter(c, q)

                @pl.when(c + NBUF < N_CHUNKS)
                def _():
                    start_gather(c + NBUF, b)

        for k in range(NTR):
            c = N_CHUNKS - NTR + k
            wait_scatter(c, c % NTR)

    return gather_kernel


_DETILE = _make_detile_kernel()
_GATHER = _make_gather_kernel()


@jax.jit
def kernel(x, table):
    tabt = jnp.swapaxes(table, 0, 1)
    tail = lax.slice(table, (TAIL_V0, 0), (VOCAB, EMBED_DIM)).reshape(32, 128)
    flat_idx = jnp.swapaxes(x, 0, 1).reshape(-1)
    tpad = _DETILE(tabt, tail)
    out5 = _GATHER(flat_idx, tpad)
    return out5.transpose(2, 4, 0, 1, 3).reshape(SEQ, TOK, EMBED_DIM)


# speedup vs baseline: 1.8680x; 1.8680x over previous
import functools
import math

import jax
import jax.numpy as jnp
from jax import lax
from jax.experimental import pallas as pl
from jax.experimental.pallas import tpu as pltpu
from jax.experimental.pallas import tpu_sc as plsc

VOCAB = 1000000
EMBED_DIM = 64
SCALE = math.sqrt(EMBED_DIM)

_INFO = plsc.get_sparse_core_info()
NC = _INFO.num_cores
NS = _INFO.num_subcores
NW = NC * NS
LANES = _INFO.num_lanes

SEQ = 4096
TOK = 200
B_TOTAL = SEQ * TOK
B_PER_W = B_TOTAL // NW
CHUNK = 256
N_CHUNKS = B_PER_W // CHUNK
NBUF = 4
LOOKAHEAD = 2
SLICES_PER_ROW = EMBED_DIM // LANES


def _make_gather_kernel():
    mesh = plsc.VectorSubcoreMesh(core_axis_name="c", subcore_axis_name="s")

    @functools.partial(
        pl.kernel,
        out_type=jax.ShapeDtypeStruct((SEQ, TOK, EMBED_DIM), jnp.float32),
        mesh=mesh,
        scratch_types=[
            pltpu.VMEM((B_PER_W,), jnp.int32),
            [pltpu.VMEM((CHUNK, EMBED_DIM), jnp.float32) for _ in range(NBUF)],
            [pltpu.SemaphoreType.DMA for _ in range(NBUF)],
            [pltpu.SemaphoreType.DMA for _ in range(NBUF)],
            pltpu.SemaphoreType.DMA,
        ],
        compiler_params=pltpu.CompilerParams(use_tc_tiling_on_sc=False),
    )
    def gather_kernel(xt_hbm, table_hbm, out_hbm, idx_v, rows, sem_g, sem_s,
                      sem_i):
        wid = lax.axis_index("s") * NC + lax.axis_index("c")
        base = wid * B_PER_W

        pltpu.async_copy(xt_hbm.at[pl.ds(base, B_PER_W)], idx_v, sem_i).wait()

        def idx_slice(c):
            return idx_v.at[pl.ds(c * CHUNK, CHUNK)]

        def out_slice(c):
            j0 = base + c * CHUNK
            t = j0 // SEQ
            s0 = j0 - t * SEQ
            return out_hbm.at[pl.ds(s0, CHUNK), t]

        def start_gather(c, b):
            pltpu.async_copy(table_hbm.at[idx_slice(c)], rows[b], sem_g[b])

        def wait_gather(c, b):
            pltpu.make_async_copy(table_hbm.at[idx_slice(c)], rows[b],
                                  sem_g[b]).wait()

        def start_scatter(c, b):
            pltpu.async_copy(rows[b], out_slice(c), sem_s[b])

        def wait_scatter(c, b):
            pltpu.make_async_copy(rows[b], out_slice(c), sem_s[b]).wait()

        for c in range(LOOKAHEAD):
            start_gather(c, c % NBUF)

        @pl.loop(0, N_CHUNKS, step=NBUF)
        def chunk_group(i):
            for db in range(NBUF):
                c = i + db
                b = db
                bn = (db + LOOKAHEAD) % NBUF

                @pl.when(c + LOOKAHEAD < N_CHUNKS)
                def _():
                    @pl.when(c >= NBUF - LOOKAHEAD)
                    def _():
                        wait_scatter(c - (NBUF - LOOKAHEAD), bn)
                    start_gather(c + LOOKAHEAD, bn)

                wait_gather(c, b)

                @plsc.parallel_loop(0, CHUNK, unroll=4)
                def scale_row(r):
                    for j in range(SLICES_PER_ROW):
                        sl = (r, pl.ds(j * LANES, LANES))
                        rows[b][sl] = rows[b][sl] * SCALE

                start_scatter(c, b)

        for k in range(NBUF):
            c = N_CHUNKS - NBUF + k
            wait_scatter(c, c % NBUF)

    return gather_kernel


_GATHER = _make_gather_kernel()


@jax.jit
def kernel(x, table):
    flat_idx = jnp.swapaxes(x, 0, 1).reshape(-1)
    return _GATHER(flat_idx, table)
